# baseline (speedup 1.0000x reference)
import functools

import jax
import jax.numpy as jnp
from jax import lax
from jax.experimental import pallas as pl
from jax.experimental.pallas import tpu as pltpu
from jax.experimental.pallas import tpu_sc as plsc

_BB = 64


def _sc_gather(neuron_list, alphas):
    (b,) = neuron_list.shape
    n = alphas.shape[1]
    info = plsc.get_sparse_core_info()
    nc, ns = info.num_cores, info.num_subcores
    nw = nc * ns
    assert b % (8 * nw) == 0
    b_per_w = b // nw

    mesh = plsc.VectorSubcoreMesh(core_axis_name="c", subcore_axis_name="s")

    @functools.partial(
        pl.kernel,
        mesh=mesh,
        out_type=jax.ShapeDtypeStruct((b, n), jnp.float32),
        scratch_types=[
            pltpu.VMEM((b_per_w,), jnp.int32),
            pltpu.VMEM((b_per_w, n), jnp.float32),
            pltpu.SemaphoreType.DMA,
        ],
        compiler_params=pltpu.CompilerParams(needs_layout_passes=False),
    )
    def k(idx_hbm, alphas_hbm, g_hbm, idx_v, rows_v, sem):
        wid = lax.axis_index("s") * nc + lax.axis_index("c")
        base = wid * b_per_w
        pltpu.sync_copy(idx_hbm.at[pl.ds(base, b_per_w)], idx_v)
        pltpu.async_copy(alphas_hbm.at[idx_v], rows_v, sem).wait()
        pltpu.sync_copy(rows_v, g_hbm.at[pl.ds(base, b_per_w)])

    return k(neuron_list, alphas)


def _tc_softmax_wsum(g, xt, cs):
    b, n = g.shape

    s = cs // 32
    half_cs = cs // 2

    def body(g_ref, x_ref, o_ref, att_ref):
        e = jnp.exp(g_ref[...])
        att_blk = e / jnp.sum(e, axis=1, keepdims=True)

        @pl.when(pl.program_id(1) == 0)
        def _():
            att_ref[...] = att_blk

        t = x_ref[...] * att_blk[:, None, None, :]
        o_ref[...] = jnp.sum(t, axis=3).reshape(_BB, half_cs)

    return pl.pallas_call(
        body,
        grid=(b // _BB, 2),
        in_specs=[
            pl.BlockSpec((_BB, n), lambda i, j: (i, 0)),
            pl.BlockSpec((_BB, 16, s, n), lambda i, j: (i, j, 0, 0)),
        ],
        out_specs=[
            pl.BlockSpec((_BB, half_cs), lambda i, j: (i, j)),
            pl.BlockSpec((_BB, n), lambda i, j: (i, 0)),
        ],
        out_shape=[
            jax.ShapeDtypeStruct((b, cs), jnp.float32),
            jax.ShapeDtypeStruct((b, n), jnp.float32),
        ],
    )(g, xt)


def kernel(x, neuron_list, alphas):
    b, n, c, s = x.shape
    cs = c * s
    xt = jnp.transpose(x, (0, 2, 3, 1))
    g = _sc_gather(neuron_list, alphas)
    out, att = _tc_softmax_wsum(g, xt, cs)
    return out, att


# speedup vs baseline: 1.0700x; 1.0700x over previous
import functools

import jax
import jax.numpy as jnp
from jax import lax
from jax.experimental import pallas as pl
from jax.experimental.pallas import tpu as pltpu
from jax.experimental.pallas import tpu_sc as plsc

_BB = 64


def _sc_gather(neuron_list, alphas):
    (b,) = neuron_list.shape
    n = alphas.shape[1]
    info = plsc.get_sparse_core_info()
    nc, ns = info.num_cores, info.num_subcores
    nw = nc * ns
    assert b % (8 * nw) == 0
    b_per_w = b // nw

    mesh = plsc.VectorSubcoreMesh(core_axis_name="c", subcore_axis_name="s")

    @functools.partial(
        pl.kernel,
        mesh=mesh,
        out_type=jax.ShapeDtypeStruct((b, n), jnp.float32),
        scratch_types=[
            pltpu.VMEM((b_per_w,), jnp.int32),
            pltpu.VMEM((b_per_w, n), jnp.float32),
            pltpu.SemaphoreType.DMA,
        ],
        compiler_params=pltpu.CompilerParams(needs_layout_passes=False),
    )
    def k(idx_hbm, alphas_hbm, g_hbm, idx_v, rows_v, sem):
        wid = lax.axis_index("s") * nc + lax.axis_index("c")
        base = wid * b_per_w
        pltpu.sync_copy(idx_hbm.at[pl.ds(base, b_per_w)], idx_v)
        pltpu.async_copy(alphas_hbm.at[idx_v], rows_v, sem).wait()
        pltpu.sync_copy(rows_v, g_hbm.at[pl.ds(base, b_per_w)])

    return k(neuron_list, alphas)


def _tc_softmax_wsum(g, xt, cs):
    b, n = g.shape

    def body(g_ref, x_ref, o_ref, att_ref):
        e = jnp.exp(g_ref[...])
        att_blk = e / jnp.sum(e, axis=1, keepdims=True)
        att_ref[...] = att_blk
        t = x_ref[...] * att_blk[:, None, None, :]
        o_ref[...] = jnp.sum(t, axis=3).reshape(_BB, cs)

    return pl.pallas_call(
        body,
        grid=(b // _BB,),
        in_specs=[
            pl.BlockSpec((_BB, n), lambda i: (i, 0)),
            pl.BlockSpec((_BB, 32, cs // 32, n), lambda i: (i, 0, 0, 0)),
        ],
        out_specs=[
            pl.BlockSpec((_BB, cs), lambda i: (i, 0)),
            pl.BlockSpec((_BB, n), lambda i: (i, 0)),
        ],
        out_shape=[
            jax.ShapeDtypeStruct((b, cs), jnp.float32),
            jax.ShapeDtypeStruct((b, n), jnp.float32),
        ],
    )(g, xt)


def kernel(x, neuron_list, alphas):
    b, n, c, s = x.shape
    cs = c * s
    xt = jnp.transpose(x, (0, 2, 3, 1))
    g = _sc_gather(neuron_list, alphas)
    out, att = _tc_softmax_wsum(g, xt, cs)
    return out, att
